# Initial kernel scaffold; baseline (speedup 1.0000x reference)
import jax
import jax.numpy as jnp
from jax.experimental import pallas as pl


def kernel(x, n_experts, cls_W1, cls_b1, cls_W2, cls_b2, we_W1, we_b1, we_W2, we_b2, ew_W1, ew_b1, ew_W2, ew_b2):
    raise NotImplementedError("write your pallas kernel here")


# speedup vs baseline: 1.2042x; 1.2042x over previous
import functools

import jax
import jax.numpy as jnp
from jax.experimental import pallas as pl
from jax.experimental.pallas import tpu as pltpu

B, D, E, C = 16384, 768, 64, 6


def _body(x_ref, n_ref, w1_ref, b1_ref, wc_ref, bc_ref, wwe_ref, bwe_ref,
          wew_ref, bew_ref, o_ref):
    x = x_ref[...]
    h = jnp.dot(x, w1_ref[...], preferred_element_type=jnp.float32) + b1_ref[...]
    h = jax.nn.gelu(h)
    h_cls = h[:, :D]
    h_we = h[:, D:2 * D]
    h_ew = h[:, 2 * D:]

    we = jnp.dot(h_we, wwe_ref[...], preferred_element_type=jnp.float32) + bwe_ref[...]
    ew = jnp.dot(h_ew, wew_ref[...], preferred_element_type=jnp.float32) + bew_ref[...]
    cls = jnp.dot(h_cls, wc_ref[...], preferred_element_type=jnp.float32) + bc_ref[...]

    g = jnp.zeros(we.shape, dtype=jnp.int32)
    for k in range(E):
        g = g + (we[:, k:k + 1] > we).astype(jnp.int32)
    n = n_ref[...]
    n_eff = jnp.where(n < 1, E, jnp.minimum(n, E))
    keep = g < n_eff

    ewm = jnp.where(keep, ew, -jnp.inf)
    m = jnp.max(ewm, axis=1, keepdims=True)
    w = jnp.exp(ewm - m)
    wsum = jnp.sum(w, axis=1, keepdims=True)

    cs = [cls[:, c * E:(c + 1) * E] for c in range(C)]
    mx = cs[0]
    for c in range(1, C):
        mx = jnp.maximum(mx, cs[c])
    es = [jnp.exp(cc - mx) for cc in cs]
    z = es[0]
    for c in range(1, C):
        z = z + es[c]
    coef = w / (z * wsum)
    outs = [jnp.sum(coef * es[c], axis=1, keepdims=True) for c in range(C)]
    o_ref[...] = jnp.concatenate(outs, axis=1)


@functools.partial(jax.jit, static_argnames=("interpret",))
def _run(x, n2, W1, b1, Wc, bc, Wwe, bwe, Wew, bew, interpret=False):
    b = x.shape[0]
    r = min(512, b)
    grid = b // r
    full = lambda shape: pl.BlockSpec(shape, lambda i: (0, 0))
    return pl.pallas_call(
        _body,
        grid=(grid,),
        in_specs=[
            pl.BlockSpec((r, D), lambda i: (i, 0)),
            pl.BlockSpec((r, 1), lambda i: (i, 0)),
            full((D, 3 * D)),
            full((1, 3 * D)),
            full((D, C * E)),
            full((1, C * E)),
            full((D, E)),
            full((1, E)),
            full((D, E)),
            full((1, E)),
        ],
        out_specs=pl.BlockSpec((r, C), lambda i: (i, 0)),
        out_shape=jax.ShapeDtypeStruct((b, C), jnp.float32),
        interpret=interpret,
    )(x, n2, W1, b1, Wc, bc, Wwe, bwe, Wew, bew)


def kernel(x, n_experts, cls_W1, cls_b1, cls_W2, cls_b2,
           we_W1, we_b1, we_W2, we_b2, ew_W1, ew_b1, ew_W2, ew_b2,
           interpret=False):
    b = x.shape[0]
    W1 = jnp.concatenate([cls_W1, we_W1, ew_W1], axis=1)
    b1 = jnp.concatenate([cls_b1, we_b1, ew_b1], axis=0).reshape(1, 3 * D)
    Wc = cls_W2.reshape(D, E, C).transpose(0, 2, 1).reshape(D, C * E)
    bc = cls_b2.reshape(E, C).transpose(1, 0).reshape(1, C * E)
    n2 = n_experts.reshape(b, 1)
    return _run(x, n2, W1, b1, Wc, bc, we_W2, we_b2.reshape(1, E),
                ew_W2, ew_b2.reshape(1, E), interpret=interpret)
